# Initial kernel scaffold; baseline (speedup 1.0000x reference)
import jax
import jax.numpy as jnp
from jax.experimental import pallas as pl


def kernel(sim):
    raise NotImplementedError("write your pallas kernel here")


# speedup vs baseline: 14.2582x; 14.2582x over previous
import jax
import jax.numpy as jnp
from jax.experimental import pallas as pl
from jax.experimental.pallas import tpu as pltpu

K = 256
ROWS_PER_BLOCK = 8
NCOLS = 32768


def _body(x_ref, o_ref, key_ref):
    x = x_ref[...]
    i = jax.lax.bitcast_convert_type(x, jnp.int32)
    key = jnp.where(i >= 0, i, i ^ jnp.int32(0x7FFFFFFF))
    key_ref[...] = key

    sign = jnp.int32(-2147483648)

    def step(b, tb):
        bit = jax.lax.shift_left(jnp.int32(1), jnp.int32(31) - b)
        candb = tb | bit
        cand = candb ^ sign
        k = key_ref[...]
        cnt = jnp.sum((k >= cand).astype(jnp.int32), axis=1, keepdims=True)
        return jnp.where(cnt >= K, candb, tb)

    tb0 = jnp.zeros((ROWS_PER_BLOCK, 1), jnp.int32)
    tb = jax.lax.fori_loop(0, 32, step, tb0)
    t = tb ^ sign
    o_ref[...] = jnp.where(key_ref[...] >= t, jnp.float32(0.0),
                           jnp.float32(jnp.inf))


def kernel(sim):
    nrows = sim.shape[0]
    grid = (nrows // ROWS_PER_BLOCK,)
    return pl.pallas_call(
        _body,
        grid=grid,
        in_specs=[pl.BlockSpec((ROWS_PER_BLOCK, NCOLS), lambda r: (r, 0))],
        out_specs=pl.BlockSpec((ROWS_PER_BLOCK, NCOLS), lambda r: (r, 0)),
        out_shape=jax.ShapeDtypeStruct(sim.shape, jnp.float32),
        scratch_shapes=[pltpu.VMEM((ROWS_PER_BLOCK, NCOLS), jnp.int32)],
    )(sim)


# speedup vs baseline: 24.3049x; 1.7046x over previous
import jax
import jax.numpy as jnp
from jax.experimental import pallas as pl
from jax.experimental.pallas import tpu as pltpu

K = 256
ROWS_PER_BLOCK = 8
NCOLS = 32768


def _body(x_ref, o_ref, key_ref):
    x = x_ref[...]
    i = jax.lax.bitcast_convert_type(x, jnp.int32)
    key = jnp.where(i >= 0, i, i ^ jnp.int32(0x7FFFFFFF))
    key_ref[...] = key

    sign = jnp.int32(-2147483648)

    NGROUPS = 8
    GW = NCOLS // NGROUPS

    def step(b, tb):
        bit = jax.lax.shift_left(jnp.int32(1), jnp.int32(31) - b)
        candb = tb | bit
        cand = candb ^ sign
        parts = [
            jnp.sum((key_ref[:, g * GW:(g + 1) * GW] >= cand).astype(jnp.int32),
                    axis=1, keepdims=True)
            for g in range(NGROUPS)
        ]
        while len(parts) > 1:
            parts = [parts[i] + parts[i + 1] for i in range(0, len(parts), 2)]
        cnt = parts[0]
        return jnp.where(cnt >= K, candb, tb)

    tb0 = jnp.zeros((ROWS_PER_BLOCK, 1), jnp.int32)
    tb = jax.lax.fori_loop(0, 32, step, tb0)
    t = tb ^ sign
    o_ref[...] = jnp.where(key_ref[...] >= t, jnp.float32(0.0),
                           jnp.float32(jnp.inf))


def kernel(sim):
    nrows = sim.shape[0]
    grid = (nrows // ROWS_PER_BLOCK,)
    return pl.pallas_call(
        _body,
        grid=grid,
        in_specs=[pl.BlockSpec((ROWS_PER_BLOCK, NCOLS), lambda r: (r, 0))],
        out_specs=pl.BlockSpec((ROWS_PER_BLOCK, NCOLS), lambda r: (r, 0)),
        out_shape=jax.ShapeDtypeStruct(sim.shape, jnp.float32),
        scratch_shapes=[pltpu.VMEM((ROWS_PER_BLOCK, NCOLS), jnp.int32)],
    )(sim)
